# baseline (speedup 1.0000x reference)
import functools

import jax
import jax.numpy as jnp
from jax import lax
from jax.experimental import pallas as pl
from jax.experimental.pallas import tpu as pltpu
from jax.experimental.pallas import tpu_sc as plsc


def _argmax_body(hw, x_ref, o_ref):
    x = x_ref[...]
    m = jnp.max(x, axis=1, keepdims=True)
    col = lax.broadcasted_iota(jnp.int32, x.shape, 1)
    cand = jnp.where(x == m, col, hw)
    o_ref[0] = jnp.min(cand, axis=1, keepdims=True)


def _argmax_call(hm_flat, bb):
    b, hw = hm_flat.shape
    grid = b // bb
    return pl.pallas_call(
        functools.partial(_argmax_body, hw),
        grid=(grid,),
        in_specs=[pl.BlockSpec((bb, hw), lambda i: (i, 0))],
        out_specs=pl.BlockSpec((1, bb, 1), lambda i: (i, 0, 0)),
        out_shape=jax.ShapeDtypeStruct((grid, bb, 1), jnp.int32),
    )(hm_flat)


def _sc_gather_body(b, w, b_per_w, l2, n_active,
                    ind_hbm, ab_hbm, trig_hbm, out_hbm,
                    ind_v, row_idx, rows_ab, rows_tr, sem):
    ab_hbm = ab_hbm.reshape(b * 2 * w, w)
    trig_hbm = trig_hbm.reshape(b * 2 * w, w)
    info = plsc.get_sparse_core_info()
    nc = info.num_cores
    wid = lax.axis_index("s") * nc + lax.axis_index("c")

    @pl.when(wid < n_active)
    def _():
        base = wid * b_per_w
        pltpu.sync_copy(ind_hbm.at[pl.ds(base, b_per_w)],
                        ind_v.at[pl.ds(0, b_per_w)])
        lane = lax.broadcasted_iota(jnp.int32, (16,), 0)
        rows_per_sample = 2 * w
        for j in range(l2 // 16):
            v = ind_v[pl.ds(j * 16, 16)]
            pos = j * 16 + lane
            valid = pos < b_per_w
            r0 = (base + pos) * rows_per_sample + lax.shift_right_logical(
                v, w.bit_length() - 1)
            row_idx[pl.ds(j * 16, 16)] = jnp.where(valid, r0, 0)
            row_idx[pl.ds(l2 + j * 16, 16)] = jnp.where(valid, r0 + w, 0)
        pltpu.async_copy(ab_hbm.at[row_idx], rows_ab, sem).wait()
        pltpu.async_copy(trig_hbm.at[row_idx], rows_tr, sem).wait()
        pltpu.sync_copy(rows_ab.at[pl.ds(0, b_per_w)],
                        out_hbm.at[pl.ds(0 * b + base, b_per_w)])
        pltpu.sync_copy(rows_ab.at[pl.ds(l2, b_per_w)],
                        out_hbm.at[pl.ds(1 * b + base, b_per_w)])
        pltpu.sync_copy(rows_tr.at[pl.ds(0, b_per_w)],
                        out_hbm.at[pl.ds(2 * b + base, b_per_w)])
        pltpu.sync_copy(rows_tr.at[pl.ds(l2, b_per_w)],
                        out_hbm.at[pl.ds(3 * b + base, b_per_w)])


def _sc_gather_call(inds, ab_rows, trig_rows, b, w):
    nw = 32
    b_per_w = None
    for cand in range(8, b + 1, 8):
        if b % cand == 0 and (b // cand) <= nw:
            b_per_w = cand
            break
    n_active = b // b_per_w
    l2 = ((b_per_w + 15) // 16) * 16

    mesh = plsc.VectorSubcoreMesh(core_axis_name="c", subcore_axis_name="s")
    fn = functools.partial(_sc_gather_body, b, w, b_per_w, l2, n_active)
    return pl.kernel(
        fn,
        mesh=mesh,
        compiler_params=pltpu.CompilerParams(use_tc_tiling_on_sc=True),
        out_type=jax.ShapeDtypeStruct((4 * b, w), jnp.float32),
        scratch_types=[
            pltpu.VMEM((l2,), jnp.int32),
            pltpu.VMEM((2 * l2,), jnp.int32),
            pltpu.VMEM((2 * l2, w), jnp.float32),
            pltpu.VMEM((2 * l2, w), jnp.float32),
            pltpu.SemaphoreType.DMA,
        ],
    )(inds, ab_rows, trig_rows)


def _tc_gather_body(bb, w, ind_ref, *refs):
    ab_refs = refs[:bb]
    tr_refs = refs[bb:2 * bb]
    o_ref = refs[2 * bb]
    i = pl.program_id(0)
    for j in range(bb):
        s = ind_ref[i * bb + j]
        r = jnp.bitwise_and(lax.shift_right_logical(s, 7), 7)
        ab = ab_refs[j][0, :, pl.ds(r, 1), :]
        tr = tr_refs[j][0, :, pl.ds(r, 1), :]
        o_ref[0, 0, j] = ab[0, 0]
        o_ref[1, 0, j] = ab[1, 0]
        o_ref[2, 0, j] = tr[0, 0]
        o_ref[3, 0, j] = tr[1, 0]


def _tc_gather_call(inds, pred_ab, pred_trig, b, h, w):
    bb = 8
    grid = b // bb

    def feat_spec(j):
        return pl.BlockSpec(
            (1, 2, 8, w),
            lambda i, ind, j=j: (i * bb + j, 0,
                                 lax.shift_right_logical(ind[i * bb + j], 10),
                                 0))

    grid_spec = pltpu.PrefetchScalarGridSpec(
        num_scalar_prefetch=1,
        grid=(grid,),
        in_specs=[feat_spec(j) for j in range(bb)] * 2,
        out_specs=pl.BlockSpec((4, 1, bb, w), lambda i, ind: (0, i, 0, 0)),
    )
    out = pl.pallas_call(
        functools.partial(_tc_gather_body, bb, w),
        grid_spec=grid_spec,
        out_shape=jax.ShapeDtypeStruct((4, grid, bb, w), jnp.float32),
    )(inds, *([pred_ab] * bb), *([pred_trig] * bb))
    return out


def _loss_body(b, w, rows_ref, ind_ref, c_ref, t_ref, o_ref):
    col = jnp.bitwise_and(ind_ref[...], w - 1).reshape(1, b, 1)
    iot = lax.broadcasted_iota(jnp.int32, (4, b, w), 2)
    g = jnp.sum(jnp.where(iot == col, rows_ref[...], 0.0), axis=2)
    ab0 = g[0]
    ab1 = g[1]
    sin2a = g[2]
    cos2a = g[3]
    xp = c_ref[0]
    yp = c_ref[1]
    xt = t_ref[0]
    yt = t_ref[1]

    lo, hi = 1e-07, 10000000.0
    wp = jnp.clip(ab0 * 2.0, lo, hi)
    hp = jnp.clip(ab1 * 2.0, lo, hi)
    wt = jnp.clip(t_ref[2], lo, hi)
    ht = jnp.clip(t_ref[3], lo, hi)

    hyp = jnp.sqrt(sin2a * sin2a + cos2a * cos2a)
    c2 = jnp.where(hyp > 0.0, cos2a / jnp.where(hyp > 0.0, hyp, 1.0), 1.0)
    cp = jnp.sqrt(jnp.clip((1.0 + c2) * 0.5, 0.0, 1.0))
    sp_mag = jnp.sqrt(jnp.clip((1.0 - c2) * 0.5, 0.0, 1.0))
    sp = jnp.where(sin2a >= 0.0, sp_mag, -sp_mag)

    rt = t_ref[4] * (jnp.pi / 180.0)
    ct = jnp.cos(rt)
    st = jnp.sin(rt)

    ap = 0.5 * wp
    bp = 0.5 * hp
    at = 0.5 * wt
    bt = 0.5 * ht
    aap = ap * ap
    bbp = bp * bp
    aat = at * at
    bbt = bt * bt

    p00 = aap * cp * cp + bbp * sp * sp
    p11 = aap * sp * sp + bbp * cp * cp
    p01 = (aap - bbp) * cp * sp
    t00 = aat * ct * ct + bbt * st * st
    t11 = aat * st * st + bbt * ct * ct
    t01 = (aat - bbt) * ct * st

    tr = p00 * t00 + 2.0 * p01 * t01 + p11 * t11
    det_sqrt = jnp.sqrt(jnp.clip((ap * bp) * (at * bt), 0.0, None))
    whr = (aap + bbp) + (aat + bbt) - 2.0 * jnp.sqrt(
        jnp.clip(tr + 2.0 * det_sqrt, 0.0, None))
    dx = xp - xt
    dy = yp - yt
    dist = jnp.clip(dx * dx + dy * dy + whr, 0.0, None)
    loss = 1.0 - 1.0 / (1.0 + dist)
    o_ref[0, 0] = jnp.sum(loss) * (1.0 / b)


def _loss_call(rows, inds, center_t, target_t, b, w):
    return pl.pallas_call(
        functools.partial(_loss_body, b, w),
        out_specs=pl.BlockSpec(memory_space=pltpu.SMEM),
        out_shape=jax.ShapeDtypeStruct((1, 1), jnp.float32),
    )(rows, inds, center_t, target_t)


def kernel(pred_hm, pred_ab, pred_trig, pred_center, target_ellipse_xywhr):
    b, c, h, w = pred_hm.shape
    hw = h * w
    hm_flat = pred_hm.reshape(b, hw)
    inds = _argmax_call(hm_flat, bb=8).reshape(b)
    rows = _tc_gather_call(inds, pred_ab, pred_trig, b, h, w)
    loss = _loss_call(rows.reshape(4, b, w), inds, pred_center.T,
                      target_ellipse_xywhr.T, b, w)
    return loss[0, 0]


# speedup vs baseline: 2.0499x; 2.0499x over previous
import functools

import jax
import jax.numpy as jnp
from jax import lax
from jax.experimental import pallas as pl
from jax.experimental.pallas import tpu as pltpu
from jax.experimental.pallas import tpu_sc as plsc


def _argmax_body(h, w, x_ref, o_ref):
    x = x_ref[:, 0]
    m2 = jnp.max(x, axis=1)
    m = jnp.max(m2, axis=1, keepdims=True)[:, :, None]
    fi = (lax.broadcasted_iota(jnp.int32, x.shape, 1) * w
          + lax.broadcasted_iota(jnp.int32, x.shape, 2))
    cand = jnp.where(x == m, fi, h * w)
    c2 = jnp.min(cand, axis=1)
    o_ref[0] = jnp.min(c2, axis=1, keepdims=True)


def _argmax_call(pred_hm, bb):
    b, c, h, w = pred_hm.shape
    grid = b // bb
    return pl.pallas_call(
        functools.partial(_argmax_body, h, w),
        grid=(grid,),
        in_specs=[pl.BlockSpec((bb, 1, h, w), lambda i: (i, 0, 0, 0))],
        out_specs=pl.BlockSpec((1, bb, 1), lambda i: (i, 0, 0)),
        out_shape=jax.ShapeDtypeStruct((grid, bb, 1), jnp.int32),
    )(pred_hm)


def _sc_gather_body(b, hw, b_per_w, l2, n_active,
                    ind_hbm, ab_hbm, trig_hbm, out_hbm,
                    ind_v, idx_ab, g_ab, g_tr, sem):
    info = plsc.get_sparse_core_info()
    nc = info.num_cores
    wid = lax.axis_index("s") * nc + lax.axis_index("c")

    @pl.when(wid < n_active)
    def _():
        base = wid * b_per_w
        pltpu.sync_copy(ind_hbm.at[pl.ds(base, b_per_w)],
                        ind_v.at[pl.ds(0, b_per_w)])
        lane = lax.broadcasted_iota(jnp.int32, (16,), 0)
        stride = 2 * hw
        for j in range(l2 // 16):
            v = ind_v[pl.ds(j * 16, 16)]
            pos = j * 16 + lane
            valid = pos < b_per_w
            flat = (base + pos) * stride + v
            idx_ab[pl.ds(j * 16, 16)] = jnp.where(valid, flat, 0)
            idx_ab[pl.ds(l2 + j * 16, 16)] = jnp.where(valid, flat + hw, 0)
        pltpu.async_copy(ab_hbm.at[idx_ab], g_ab, sem).wait()
        pltpu.async_copy(trig_hbm.at[idx_ab], g_tr, sem).wait()
        pltpu.sync_copy(g_ab.at[pl.ds(0, b_per_w)],
                        out_hbm.at[pl.ds(0 * b + base, b_per_w)])
        pltpu.sync_copy(g_ab.at[pl.ds(l2, b_per_w)],
                        out_hbm.at[pl.ds(1 * b + base, b_per_w)])
        pltpu.sync_copy(g_tr.at[pl.ds(0, b_per_w)],
                        out_hbm.at[pl.ds(2 * b + base, b_per_w)])
        pltpu.sync_copy(g_tr.at[pl.ds(l2, b_per_w)],
                        out_hbm.at[pl.ds(3 * b + base, b_per_w)])


def _sc_gather_call(inds, ab_flat, trig_flat, b, hw):
    nw = 32
    b_per_w = None
    for cand in range(8, b + 1, 8):
        if b % cand == 0 and (b // cand) <= nw:
            b_per_w = cand
            break
    n_active = b // b_per_w
    l2 = ((b_per_w + 15) // 16) * 16

    mesh = plsc.VectorSubcoreMesh(core_axis_name="c", subcore_axis_name="s")
    fn = functools.partial(_sc_gather_body, b, hw, b_per_w, l2, n_active)
    return pl.kernel(
        fn,
        mesh=mesh,
        out_type=jax.ShapeDtypeStruct((4 * b,), jnp.float32),
        scratch_types=[
            pltpu.VMEM((l2,), jnp.int32),
            pltpu.VMEM((2 * l2,), jnp.int32),
            pltpu.VMEM((2 * l2,), jnp.float32),
            pltpu.VMEM((2 * l2,), jnp.float32),
            pltpu.SemaphoreType.DMA,
        ],
    )(inds, ab_flat, trig_flat)


def _loss_body(b, g_ref, c_ref, t_ref, o_ref):
    ab0 = g_ref[0]
    ab1 = g_ref[1]
    sin2a = g_ref[2]
    cos2a = g_ref[3]
    xp = c_ref[0]
    yp = c_ref[1]
    xt = t_ref[0]
    yt = t_ref[1]

    lo, hi = 1e-07, 10000000.0
    wp = jnp.clip(ab0 * 2.0, lo, hi)
    hp = jnp.clip(ab1 * 2.0, lo, hi)
    wt = jnp.clip(t_ref[2], lo, hi)
    ht = jnp.clip(t_ref[3], lo, hi)

    hyp = jnp.sqrt(sin2a * sin2a + cos2a * cos2a)
    c2 = jnp.where(hyp > 0.0, cos2a / jnp.where(hyp > 0.0, hyp, 1.0), 1.0)
    cp = jnp.sqrt(jnp.clip((1.0 + c2) * 0.5, 0.0, 1.0))
    sp_mag = jnp.sqrt(jnp.clip((1.0 - c2) * 0.5, 0.0, 1.0))
    sp = jnp.where(sin2a >= 0.0, sp_mag, -sp_mag)

    rt = t_ref[4] * (jnp.pi / 180.0)
    ct = jnp.cos(rt)
    st = jnp.sin(rt)

    ap = 0.5 * wp
    bp = 0.5 * hp
    at = 0.5 * wt
    bt = 0.5 * ht
    aap = ap * ap
    bbp = bp * bp
    aat = at * at
    bbt = bt * bt

    p00 = aap * cp * cp + bbp * sp * sp
    p11 = aap * sp * sp + bbp * cp * cp
    p01 = (aap - bbp) * cp * sp
    t00 = aat * ct * ct + bbt * st * st
    t11 = aat * st * st + bbt * ct * ct
    t01 = (aat - bbt) * ct * st

    tr = p00 * t00 + 2.0 * p01 * t01 + p11 * t11
    det_sqrt = jnp.sqrt(jnp.clip((ap * bp) * (at * bt), 0.0, None))
    whr = (aap + bbp) + (aat + bbt) - 2.0 * jnp.sqrt(
        jnp.clip(tr + 2.0 * det_sqrt, 0.0, None))
    dx = xp - xt
    dy = yp - yt
    dist = jnp.clip(dx * dx + dy * dy + whr, 0.0, None)
    loss = 1.0 - 1.0 / (1.0 + dist)
    o_ref[0, 0] = jnp.sum(loss) * (1.0 / b)


def _loss_call(g, center_t, target_t, b):
    return pl.pallas_call(
        functools.partial(_loss_body, b),
        out_specs=pl.BlockSpec(memory_space=pltpu.SMEM),
        out_shape=jax.ShapeDtypeStruct((1, 1), jnp.float32),
    )(g, center_t, target_t)


def kernel(pred_hm, pred_ab, pred_trig, pred_center, target_ellipse_xywhr):
    b, c, h, w = pred_hm.shape
    hw = h * w
    inds = _argmax_call(pred_hm, bb=8).reshape(b)
    g = _sc_gather_call(inds, pred_ab.reshape(b * 2 * hw),
                        pred_trig.reshape(b * 2 * hw), b, hw).reshape(4, b)
    loss = _loss_call(g, pred_center.T, target_ellipse_xywhr.T, b)
    return loss[0, 0]


# speedup vs baseline: 4.1194x; 2.0096x over previous
import functools

import jax
import jax.numpy as jnp
from jax import lax
from jax.experimental import pallas as pl
from jax.experimental.pallas import tpu as pltpu
from jax.experimental.pallas import tpu_sc as plsc


def _argmax_body(h, w, x_ref, o_ref):
    x = x_ref[:, 0]
    m2 = jnp.max(x, axis=1)
    m = jnp.max(m2, axis=1, keepdims=True)[:, :, None]
    fi = (lax.broadcasted_iota(jnp.int32, x.shape, 1) * w
          + lax.broadcasted_iota(jnp.int32, x.shape, 2))
    cand = jnp.where(x == m, fi, h * w)
    c2 = jnp.min(cand, axis=1)
    o_ref[0] = jnp.min(c2, axis=1, keepdims=True)


def _argmax_call(pred_hm, bb):
    b, c, h, w = pred_hm.shape
    grid = b // bb
    return pl.pallas_call(
        functools.partial(_argmax_body, h, w),
        grid=(grid,),
        in_specs=[pl.BlockSpec((bb, 1, h, w), lambda i: (i, 0, 0, 0))],
        out_specs=pl.BlockSpec((1, bb, 1), lambda i: (i, 0, 0)),
        out_shape=jax.ShapeDtypeStruct((grid, bb, 1), jnp.int32),
    )(pred_hm)


def _sc_gather_body(b, hw, b_per_w, l2, n_active,
                    ind_hbm, ab_hbm, trig_hbm, out_hbm,
                    ind_v, idx_ab, g_ab, g_tr, sem):
    info = plsc.get_sparse_core_info()
    nc = info.num_cores
    wid = lax.axis_index("s") * nc + lax.axis_index("c")

    @pl.when(wid < n_active)
    def _():
        base = wid * b_per_w
        pltpu.sync_copy(ind_hbm.at[pl.ds(base, b_per_w)],
                        ind_v.at[pl.ds(0, b_per_w)])
        lane = lax.broadcasted_iota(jnp.int32, (16,), 0)
        stride = 2 * hw
        for j in range(l2 // 16):
            v = ind_v[pl.ds(j * 16, 16)]
            pos = j * 16 + lane
            valid = pos < b_per_w
            flat = (base + pos) * stride + v
            idx_ab[pl.ds(j * 16, 16)] = jnp.where(valid, flat, 0)
            idx_ab[pl.ds(l2 + j * 16, 16)] = jnp.where(valid, flat + hw, 0)
        pltpu.async_copy(ab_hbm.at[idx_ab], g_ab, sem).wait()
        pltpu.async_copy(trig_hbm.at[idx_ab], g_tr, sem).wait()
        pltpu.sync_copy(g_ab.at[pl.ds(0, b_per_w)],
                        out_hbm.at[pl.ds(0 * b + base, b_per_w)])
        pltpu.sync_copy(g_ab.at[pl.ds(l2, b_per_w)],
                        out_hbm.at[pl.ds(1 * b + base, b_per_w)])
        pltpu.sync_copy(g_tr.at[pl.ds(0, b_per_w)],
                        out_hbm.at[pl.ds(2 * b + base, b_per_w)])
        pltpu.sync_copy(g_tr.at[pl.ds(l2, b_per_w)],
                        out_hbm.at[pl.ds(3 * b + base, b_per_w)])


def _sc_gather_call(inds, ab_flat, trig_flat, b, hw):
    nw = 32
    b_per_w = None
    for cand in range(8, b + 1, 8):
        if b % cand == 0 and (b // cand) <= nw:
            b_per_w = cand
            break
    n_active = b // b_per_w
    l2 = ((b_per_w + 15) // 16) * 16

    mesh = plsc.VectorSubcoreMesh(core_axis_name="c", subcore_axis_name="s")
    fn = functools.partial(_sc_gather_body, b, hw, b_per_w, l2, n_active)
    return pl.kernel(
        fn,
        mesh=mesh,
        out_type=jax.ShapeDtypeStruct((4 * b,), jnp.float32),
        scratch_types=[
            pltpu.VMEM((l2,), jnp.int32),
            pltpu.VMEM((2 * l2,), jnp.int32),
            pltpu.VMEM((2 * l2,), jnp.float32),
            pltpu.VMEM((2 * l2,), jnp.float32),
            pltpu.SemaphoreType.DMA,
        ],
    )(inds, ab_flat, trig_flat)


def _loss_body(b, g_ref, c_ref, t_ref, o_ref):
    ab0 = g_ref[0]
    ab1 = g_ref[1]
    sin2a = g_ref[2]
    cos2a = g_ref[3]
    xp = c_ref[0]
    yp = c_ref[1]
    xt = t_ref[0]
    yt = t_ref[1]

    lo, hi = 1e-07, 10000000.0
    wp = jnp.clip(ab0 * 2.0, lo, hi)
    hp = jnp.clip(ab1 * 2.0, lo, hi)
    wt = jnp.clip(t_ref[2], lo, hi)
    ht = jnp.clip(t_ref[3], lo, hi)

    hyp = jnp.sqrt(sin2a * sin2a + cos2a * cos2a)
    c2 = jnp.where(hyp > 0.0, cos2a / jnp.where(hyp > 0.0, hyp, 1.0), 1.0)
    cp = jnp.sqrt(jnp.clip((1.0 + c2) * 0.5, 0.0, 1.0))
    sp_mag = jnp.sqrt(jnp.clip((1.0 - c2) * 0.5, 0.0, 1.0))
    sp = jnp.where(sin2a >= 0.0, sp_mag, -sp_mag)

    rt = t_ref[4] * (jnp.pi / 180.0)
    ct = jnp.cos(rt)
    st = jnp.sin(rt)

    ap = 0.5 * wp
    bp = 0.5 * hp
    at = 0.5 * wt
    bt = 0.5 * ht
    aap = ap * ap
    bbp = bp * bp
    aat = at * at
    bbt = bt * bt

    p00 = aap * cp * cp + bbp * sp * sp
    p11 = aap * sp * sp + bbp * cp * cp
    p01 = (aap - bbp) * cp * sp
    t00 = aat * ct * ct + bbt * st * st
    t11 = aat * st * st + bbt * ct * ct
    t01 = (aat - bbt) * ct * st

    tr = p00 * t00 + 2.0 * p01 * t01 + p11 * t11
    det_sqrt = jnp.sqrt(jnp.clip((ap * bp) * (at * bt), 0.0, None))
    whr = (aap + bbp) + (aat + bbt) - 2.0 * jnp.sqrt(
        jnp.clip(tr + 2.0 * det_sqrt, 0.0, None))
    dx = xp - xt
    dy = yp - yt
    dist = jnp.clip(dx * dx + dy * dy + whr, 0.0, None)
    loss = 1.0 - 1.0 / (1.0 + dist)
    o_ref[0, 0] = jnp.sum(loss) * (1.0 / b)


def _loss_call(g, center_t, target_t, b):
    return pl.pallas_call(
        functools.partial(_loss_body, b),
        out_specs=pl.BlockSpec(memory_space=pltpu.SMEM),
        out_shape=jax.ShapeDtypeStruct((1, 1), jnp.float32),
    )(g, center_t, target_t)


def kernel(pred_hm, pred_ab, pred_trig, pred_center, target_ellipse_xywhr):
    b, c, h, w = pred_hm.shape
    hw = h * w
    inds = _argmax_call(pred_hm, bb=40).reshape(b)
    g = _sc_gather_call(inds, pred_ab.reshape(b * 2 * hw),
                        pred_trig.reshape(b * 2 * hw), b, hw).reshape(4, b)
    loss = _loss_call(g, pred_center.T, target_ellipse_xywhr.T, b)
    return loss[0, 0]


# speedup vs baseline: 4.9110x; 1.1922x over previous
import functools

import jax
import jax.numpy as jnp
from jax import lax
from jax.experimental import pallas as pl
from jax.experimental.pallas import tpu as pltpu
from jax.experimental.pallas import tpu_sc as plsc


def _argmax_body(h, w, x_ref, o_ref):
    x = x_ref[:, 0]
    m2 = jnp.max(x, axis=1)
    m = jnp.max(m2, axis=1, keepdims=True)[:, :, None]
    fi = (lax.broadcasted_iota(jnp.int32, x.shape, 1) * w
          + lax.broadcasted_iota(jnp.int32, x.shape, 2))
    cand = jnp.where(x == m, fi, h * w)
    c2 = jnp.min(cand, axis=1)
    o_ref[0] = jnp.min(c2, axis=1, keepdims=True)


def _argmax_call(pred_hm, bb):
    b, c, h, w = pred_hm.shape
    grid = b // bb
    return pl.pallas_call(
        functools.partial(_argmax_body, h, w),
        grid=(grid,),
        in_specs=[pl.BlockSpec((bb, 1, h, w), lambda i: (i, 0, 0, 0))],
        out_specs=pl.BlockSpec((1, bb, 1), lambda i: (i, 0, 0)),
        out_shape=jax.ShapeDtypeStruct((grid, bb, 1), jnp.int32),
    )(pred_hm)


def _sc_gather_body(b, hw, b_per_w, l2, n_active,
                    ind_hbm, ab_hbm, trig_hbm, out_hbm,
                    ind_v, idx_ab, g_ab, g_tr, sem):
    info = plsc.get_sparse_core_info()
    nc = info.num_cores
    wid = lax.axis_index("s") * nc + lax.axis_index("c")

    @pl.when(wid < n_active)
    def _():
        base = wid * b_per_w
        pltpu.sync_copy(ind_hbm.at[pl.ds(base, b_per_w)],
                        ind_v.at[pl.ds(0, b_per_w)])
        lane = lax.broadcasted_iota(jnp.int32, (16,), 0)
        stride = 2 * hw
        for j in range(l2 // 16):
            v = ind_v[pl.ds(j * 16, 16)]
            pos = j * 16 + lane
            valid = pos < b_per_w
            flat = (base + pos) * stride + v
            idx_ab[pl.ds(j * 16, 16)] = jnp.where(valid, flat, 0)
            idx_ab[pl.ds(l2 + j * 16, 16)] = jnp.where(valid, flat + hw, 0)
        pltpu.async_copy(ab_hbm.at[idx_ab], g_ab, sem).wait()
        pltpu.async_copy(trig_hbm.at[idx_ab], g_tr, sem).wait()
        pltpu.sync_copy(g_ab.at[pl.ds(0, b_per_w)],
                        out_hbm.at[pl.ds(0 * b + base, b_per_w)])
        pltpu.sync_copy(g_ab.at[pl.ds(l2, b_per_w)],
                        out_hbm.at[pl.ds(1 * b + base, b_per_w)])
        pltpu.sync_copy(g_tr.at[pl.ds(0, b_per_w)],
                        out_hbm.at[pl.ds(2 * b + base, b_per_w)])
        pltpu.sync_copy(g_tr.at[pl.ds(l2, b_per_w)],
                        out_hbm.at[pl.ds(3 * b + base, b_per_w)])


def _sc_gather_call(inds, ab_flat, trig_flat, b, hw):
    nw = 32
    b_per_w = None
    for cand in range(8, b + 1, 8):
        if b % cand == 0 and (b // cand) <= nw:
            b_per_w = cand
            break
    n_active = b // b_per_w
    l2 = ((b_per_w + 15) // 16) * 16

    mesh = plsc.VectorSubcoreMesh(core_axis_name="c", subcore_axis_name="s")
    fn = functools.partial(_sc_gather_body, b, hw, b_per_w, l2, n_active)
    return pl.kernel(
        fn,
        mesh=mesh,
        out_type=jax.ShapeDtypeStruct((4 * b,), jnp.float32),
        scratch_types=[
            pltpu.VMEM((l2,), jnp.int32),
            pltpu.VMEM((2 * l2,), jnp.int32),
            pltpu.VMEM((2 * l2,), jnp.float32),
            pltpu.VMEM((2 * l2,), jnp.float32),
            pltpu.SemaphoreType.DMA,
        ],
    )(inds, ab_flat, trig_flat)


def _loss_body(b, g_ref, c_ref, t_ref, o_ref):
    ab0 = g_ref[0]
    ab1 = g_ref[1]
    sin2a = g_ref[2]
    cos2a = g_ref[3]
    xp = c_ref[0]
    yp = c_ref[1]
    xt = t_ref[0]
    yt = t_ref[1]

    lo, hi = 1e-07, 10000000.0
    wp = jnp.clip(ab0 * 2.0, lo, hi)
    hp = jnp.clip(ab1 * 2.0, lo, hi)
    wt = jnp.clip(t_ref[2], lo, hi)
    ht = jnp.clip(t_ref[3], lo, hi)

    hyp = jnp.sqrt(sin2a * sin2a + cos2a * cos2a)
    c2 = jnp.where(hyp > 0.0, cos2a / jnp.where(hyp > 0.0, hyp, 1.0), 1.0)
    cp = jnp.sqrt(jnp.clip((1.0 + c2) * 0.5, 0.0, 1.0))
    sp_mag = jnp.sqrt(jnp.clip((1.0 - c2) * 0.5, 0.0, 1.0))
    sp = jnp.where(sin2a >= 0.0, sp_mag, -sp_mag)

    rt = t_ref[4] * (jnp.pi / 180.0)
    ct = jnp.cos(rt)
    st = jnp.sin(rt)

    ap = 0.5 * wp
    bp = 0.5 * hp
    at = 0.5 * wt
    bt = 0.5 * ht
    aap = ap * ap
    bbp = bp * bp
    aat = at * at
    bbt = bt * bt

    p00 = aap * cp * cp + bbp * sp * sp
    p11 = aap * sp * sp + bbp * cp * cp
    p01 = (aap - bbp) * cp * sp
    t00 = aat * ct * ct + bbt * st * st
    t11 = aat * st * st + bbt * ct * ct
    t01 = (aat - bbt) * ct * st

    tr = p00 * t00 + 2.0 * p01 * t01 + p11 * t11
    det_sqrt = jnp.sqrt(jnp.clip((ap * bp) * (at * bt), 0.0, None))
    whr = (aap + bbp) + (aat + bbt) - 2.0 * jnp.sqrt(
        jnp.clip(tr + 2.0 * det_sqrt, 0.0, None))
    dx = xp - xt
    dy = yp - yt
    dist = jnp.clip(dx * dx + dy * dy + whr, 0.0, None)
    loss = 1.0 - 1.0 / (1.0 + dist)
    o_ref[0, 0] = jnp.sum(loss) * (1.0 / b)


def _loss_call(g, center_t, target_t, b):
    return pl.pallas_call(
        functools.partial(_loss_body, b),
        out_specs=pl.BlockSpec(memory_space=pltpu.SMEM),
        out_shape=jax.ShapeDtypeStruct((1, 1), jnp.float32),
    )(g, center_t, target_t)


def kernel(pred_hm, pred_ab, pred_trig, pred_center, target_ellipse_xywhr):
    b, c, h, w = pred_hm.shape
    hw = h * w
    inds = _argmax_call(pred_hm, bb=125).reshape(b)
    g = _sc_gather_call(inds, pred_ab.reshape(b * 2 * hw),
                        pred_trig.reshape(b * 2 * hw), b, hw).reshape(4, b)
    loss = _loss_call(g, pred_center.T, target_ellipse_xywhr.T, b)
    return loss[0, 0]


# speedup vs baseline: 5.0200x; 1.0222x over previous
import functools

import jax
import jax.numpy as jnp
from jax import lax
from jax.experimental import pallas as pl
from jax.experimental.pallas import tpu as pltpu
from jax.experimental.pallas import tpu_sc as plsc


def _argmax_body(h, w, x_ref, o_ref):
    x = x_ref[:, 0]
    m2 = jnp.max(x, axis=1)
    m = jnp.max(m2, axis=1, keepdims=True)[:, :, None]
    fi = (lax.broadcasted_iota(jnp.int32, x.shape, 1) * w
          + lax.broadcasted_iota(jnp.int32, x.shape, 2))
    cand = jnp.where(x == m, fi, h * w)
    c2 = jnp.min(cand, axis=1)
    o_ref[0] = jnp.min(c2, axis=1, keepdims=True)


def _argmax_call(pred_hm, bb):
    b, c, h, w = pred_hm.shape
    grid = b // bb
    return pl.pallas_call(
        functools.partial(_argmax_body, h, w),
        grid=(grid,),
        in_specs=[pl.BlockSpec((bb, 1, h, w), lambda i: (i, 0, 0, 0))],
        out_specs=pl.BlockSpec((1, bb, 1), lambda i: (i, 0, 0)),
        out_shape=jax.ShapeDtypeStruct((grid, bb, 1), jnp.int32),
    )(pred_hm)


def _sc_gather_body(b, hw, b_per_w, l2, n_active,
                    ind_hbm, ab_hbm, trig_hbm, out_hbm,
                    ind_v, idx_ab, g_ab, g_tr, sem):
    info = plsc.get_sparse_core_info()
    nc = info.num_cores
    wid = lax.axis_index("s") * nc + lax.axis_index("c")

    @pl.when(wid < n_active)
    def _():
        base = wid * b_per_w
        pltpu.sync_copy(ind_hbm.at[pl.ds(base, b_per_w)],
                        ind_v.at[pl.ds(0, b_per_w)])
        lane = lax.broadcasted_iota(jnp.int32, (16,), 0)
        stride = 2 * hw
        for j in range(l2 // 16):
            v = ind_v[pl.ds(j * 16, 16)]
            pos = j * 16 + lane
            valid = pos < b_per_w
            flat = (base + pos) * stride + v
            idx_ab[pl.ds(j * 16, 16)] = jnp.where(valid, flat, 0)
            idx_ab[pl.ds(l2 + j * 16, 16)] = jnp.where(valid, flat + hw, 0)
        pltpu.async_copy(ab_hbm.at[idx_ab], g_ab, sem).wait()
        pltpu.async_copy(trig_hbm.at[idx_ab], g_tr, sem).wait()
        pltpu.sync_copy(g_ab.at[pl.ds(0, b_per_w)],
                        out_hbm.at[pl.ds(0 * b + base, b_per_w)])
        pltpu.sync_copy(g_ab.at[pl.ds(l2, b_per_w)],
                        out_hbm.at[pl.ds(1 * b + base, b_per_w)])
        pltpu.sync_copy(g_tr.at[pl.ds(0, b_per_w)],
                        out_hbm.at[pl.ds(2 * b + base, b_per_w)])
        pltpu.sync_copy(g_tr.at[pl.ds(l2, b_per_w)],
                        out_hbm.at[pl.ds(3 * b + base, b_per_w)])


def _sc_gather_call(inds, ab_flat, trig_flat, b, hw):
    nw = 32
    b_per_w = None
    for cand in range(8, b + 1, 8):
        if b % cand == 0 and (b // cand) <= nw:
            b_per_w = cand
            break
    n_active = b // b_per_w
    l2 = ((b_per_w + 15) // 16) * 16

    mesh = plsc.VectorSubcoreMesh(core_axis_name="c", subcore_axis_name="s")
    fn = functools.partial(_sc_gather_body, b, hw, b_per_w, l2, n_active)
    return pl.kernel(
        fn,
        mesh=mesh,
        out_type=jax.ShapeDtypeStruct((4 * b,), jnp.float32),
        scratch_types=[
            pltpu.VMEM((l2,), jnp.int32),
            pltpu.VMEM((2 * l2,), jnp.int32),
            pltpu.VMEM((2 * l2,), jnp.float32),
            pltpu.VMEM((2 * l2,), jnp.float32),
            pltpu.SemaphoreType.DMA,
        ],
    )(inds, ab_flat, trig_flat)


def _loss_body(b, g_ref, c_ref, t_ref, o_ref):
    ab0 = g_ref[pl.ds(0 * b, b)]
    ab1 = g_ref[pl.ds(1 * b, b)]
    sin2a = g_ref[pl.ds(2 * b, b)]
    cos2a = g_ref[pl.ds(3 * b, b)]
    xp = c_ref[0]
    yp = c_ref[1]
    xt = t_ref[0]
    yt = t_ref[1]

    lo, hi = 1e-07, 10000000.0
    wp = jnp.clip(ab0 * 2.0, lo, hi)
    hp = jnp.clip(ab1 * 2.0, lo, hi)
    wt = jnp.clip(t_ref[2], lo, hi)
    ht = jnp.clip(t_ref[3], lo, hi)

    hyp = jnp.sqrt(sin2a * sin2a + cos2a * cos2a)
    c2 = jnp.where(hyp > 0.0, cos2a / jnp.where(hyp > 0.0, hyp, 1.0), 1.0)
    cp = jnp.sqrt(jnp.clip((1.0 + c2) * 0.5, 0.0, 1.0))
    sp_mag = jnp.sqrt(jnp.clip((1.0 - c2) * 0.5, 0.0, 1.0))
    sp = jnp.where(sin2a >= 0.0, sp_mag, -sp_mag)

    rt = t_ref[4] * (jnp.pi / 180.0)
    ct = jnp.cos(rt)
    st = jnp.sin(rt)

    ap = 0.5 * wp
    bp = 0.5 * hp
    at = 0.5 * wt
    bt = 0.5 * ht
    aap = ap * ap
    bbp = bp * bp
    aat = at * at
    bbt = bt * bt

    p00 = aap * cp * cp + bbp * sp * sp
    p11 = aap * sp * sp + bbp * cp * cp
    p01 = (aap - bbp) * cp * sp
    t00 = aat * ct * ct + bbt * st * st
    t11 = aat * st * st + bbt * ct * ct
    t01 = (aat - bbt) * ct * st

    tr = p00 * t00 + 2.0 * p01 * t01 + p11 * t11
    det_sqrt = jnp.sqrt(jnp.clip((ap * bp) * (at * bt), 0.0, None))
    whr = (aap + bbp) + (aat + bbt) - 2.0 * jnp.sqrt(
        jnp.clip(tr + 2.0 * det_sqrt, 0.0, None))
    dx = xp - xt
    dy = yp - yt
    dist = jnp.clip(dx * dx + dy * dy + whr, 0.0, None)
    loss = 1.0 - 1.0 / (1.0 + dist)
    o_ref[0, 0] = jnp.sum(loss) * (1.0 / b)


def _loss_call(g_flat, center, target, b):
    return pl.pallas_call(
        functools.partial(_loss_body, b),
        out_specs=pl.BlockSpec(memory_space=pltpu.SMEM),
        out_shape=jax.ShapeDtypeStruct((1, 1), jnp.float32),
    )(g_flat, center, target)


def kernel(pred_hm, pred_ab, pred_trig, pred_center, target_ellipse_xywhr):
    b, c, h, w = pred_hm.shape
    hw = h * w
    inds = _argmax_call(pred_hm, bb=125).reshape(b)
    g = _sc_gather_call(inds, pred_ab.reshape(b * 2 * hw),
                        pred_trig.reshape(b * 2 * hw), b, hw)
    loss = _loss_call(g, pred_center.T, target_ellipse_xywhr.T, b)
    return loss[0, 0]


# speedup vs baseline: 5.0326x; 1.0025x over previous
import functools

import jax
import jax.numpy as jnp
from jax import lax
from jax.experimental import pallas as pl
from jax.experimental.pallas import tpu as pltpu
from jax.experimental.pallas import tpu_sc as plsc


def _argmax_body(h, w, x_ref, o_ref):
    x = x_ref[:, 0]
    m2 = jnp.max(x, axis=1)
    m = jnp.max(m2, axis=1, keepdims=True)[:, :, None]
    fi = (lax.broadcasted_iota(jnp.int32, x.shape, 1) * w
          + lax.broadcasted_iota(jnp.int32, x.shape, 2))
    cand = jnp.where(x == m, fi, h * w)
    c2 = jnp.min(cand, axis=1)
    o_ref[0] = jnp.min(c2, axis=1, keepdims=True)


def _argmax_call(pred_hm, bb):
    b, c, h, w = pred_hm.shape
    grid = b // bb
    return pl.pallas_call(
        functools.partial(_argmax_body, h, w),
        grid=(grid,),
        in_specs=[pl.BlockSpec((bb, 1, h, w), lambda i: (i, 0, 0, 0))],
        out_specs=pl.BlockSpec((1, bb, 1), lambda i: (i, 0, 0)),
        out_shape=jax.ShapeDtypeStruct((grid, bb, 1), jnp.int32),
    )(pred_hm)


def _sc_gather_body(b, hw, b_per_w, l2, n_active,
                    ind_hbm, ab_hbm, trig_hbm, out_hbm,
                    ind_v, idx_ab, g_ab, g_tr, sem):
    info = plsc.get_sparse_core_info()
    nc = info.num_cores
    wid = lax.axis_index("s") * nc + lax.axis_index("c")

    @pl.when(wid < n_active)
    def _():
        base = wid * b_per_w
        pltpu.sync_copy(ind_hbm.at[pl.ds(base, b_per_w)],
                        ind_v.at[pl.ds(0, b_per_w)])
        lane = lax.broadcasted_iota(jnp.int32, (16,), 0)
        stride = 2 * hw
        for j in range(l2 // 16):
            v = ind_v[pl.ds(j * 16, 16)]
            pos = j * 16 + lane
            valid = pos < b_per_w
            flat = (base + pos) * stride + v
            idx_ab[pl.ds(j * 16, 16)] = jnp.where(valid, flat, 0)
            idx_ab[pl.ds(l2 + j * 16, 16)] = jnp.where(valid, flat + hw, 0)
        pltpu.async_copy(ab_hbm.at[idx_ab], g_ab, sem).wait()
        pltpu.async_copy(trig_hbm.at[idx_ab], g_tr, sem).wait()
        pltpu.sync_copy(g_ab.at[pl.ds(0, b_per_w)],
                        out_hbm.at[pl.ds(0 * b + base, b_per_w)])
        pltpu.sync_copy(g_ab.at[pl.ds(l2, b_per_w)],
                        out_hbm.at[pl.ds(1 * b + base, b_per_w)])
        pltpu.sync_copy(g_tr.at[pl.ds(0, b_per_w)],
                        out_hbm.at[pl.ds(2 * b + base, b_per_w)])
        pltpu.sync_copy(g_tr.at[pl.ds(l2, b_per_w)],
                        out_hbm.at[pl.ds(3 * b + base, b_per_w)])


def _sc_gather_call(inds, ab_flat, trig_flat, b, hw):
    nw = 32
    b_per_w = None
    for cand in range(8, b + 1, 8):
        if b % cand == 0 and (b // cand) <= nw:
            b_per_w = cand
            break
    n_active = b // b_per_w
    l2 = ((b_per_w + 15) // 16) * 16

    mesh = plsc.VectorSubcoreMesh(core_axis_name="c", subcore_axis_name="s")
    fn = functools.partial(_sc_gather_body, b, hw, b_per_w, l2, n_active)
    return pl.kernel(
        fn,
        mesh=mesh,
        out_type=jax.ShapeDtypeStruct((4 * b,), jnp.float32),
        scratch_types=[
            pltpu.VMEM((l2,), jnp.int32),
            pltpu.VMEM((2 * l2,), jnp.int32),
            pltpu.VMEM((2 * l2,), jnp.float32),
            pltpu.VMEM((2 * l2,), jnp.float32),
            pltpu.SemaphoreType.DMA,
        ],
    )(inds, ab_flat, trig_flat)


def _loss_body(b, g_ref, c_ref, t_ref, o_ref):
    ab0 = g_ref[pl.ds(0 * b, b)]
    ab1 = g_ref[pl.ds(1 * b, b)]
    sin2a = g_ref[pl.ds(2 * b, b)]
    cos2a = g_ref[pl.ds(3 * b, b)]
    xp = c_ref[0]
    yp = c_ref[1]
    xt = t_ref[0]
    yt = t_ref[1]

    lo, hi = 1e-07, 10000000.0
    wp = jnp.clip(ab0 * 2.0, lo, hi)
    hp = jnp.clip(ab1 * 2.0, lo, hi)
    wt = jnp.clip(t_ref[2], lo, hi)
    ht = jnp.clip(t_ref[3], lo, hi)

    hyp = jnp.sqrt(sin2a * sin2a + cos2a * cos2a)
    c2 = jnp.where(hyp > 0.0, cos2a / jnp.where(hyp > 0.0, hyp, 1.0), 1.0)
    cp = jnp.sqrt(jnp.clip((1.0 + c2) * 0.5, 0.0, 1.0))
    sp_mag = jnp.sqrt(jnp.clip((1.0 - c2) * 0.5, 0.0, 1.0))
    sp = jnp.where(sin2a >= 0.0, sp_mag, -sp_mag)

    rt = t_ref[4] * (jnp.pi / 180.0)
    ct = jnp.cos(rt)
    st = jnp.sin(rt)

    ap = 0.5 * wp
    bp = 0.5 * hp
    at = 0.5 * wt
    bt = 0.5 * ht
    aap = ap * ap
    bbp = bp * bp
    aat = at * at
    bbt = bt * bt

    p00 = aap * cp * cp + bbp * sp * sp
    p11 = aap * sp * sp + bbp * cp * cp
    p01 = (aap - bbp) * cp * sp
    t00 = aat * ct * ct + bbt * st * st
    t11 = aat * st * st + bbt * ct * ct
    t01 = (aat - bbt) * ct * st

    tr = p00 * t00 + 2.0 * p01 * t01 + p11 * t11
    det_sqrt = jnp.sqrt(jnp.clip((ap * bp) * (at * bt), 0.0, None))
    whr = (aap + bbp) + (aat + bbt) - 2.0 * jnp.sqrt(
        jnp.clip(tr + 2.0 * det_sqrt, 0.0, None))
    dx = xp - xt
    dy = yp - yt
    dist = jnp.clip(dx * dx + dy * dy + whr, 0.0, None)
    loss = 1.0 - 1.0 / (1.0 + dist)
    o_ref[0, 0] = jnp.sum(loss) * (1.0 / b)


def _loss_call(g_flat, center, target, b):
    return pl.pallas_call(
        functools.partial(_loss_body, b),
        out_specs=pl.BlockSpec(memory_space=pltpu.SMEM),
        out_shape=jax.ShapeDtypeStruct((1, 1), jnp.float32),
    )(g_flat, center, target)


def kernel(pred_hm, pred_ab, pred_trig, pred_center, target_ellipse_xywhr):
    b, c, h, w = pred_hm.shape
    hw = h * w
    inds = _argmax_call(pred_hm, bb=250).reshape(b)
    g = _sc_gather_call(inds, pred_ab.reshape(b * 2 * hw),
                        pred_trig.reshape(b * 2 * hw), b, hw)
    loss = _loss_call(g, pred_center.T, target_ellipse_xywhr.T, b)
    return loss[0, 0]
